# Initial kernel scaffold; baseline (speedup 1.0000x reference)
import jax
import jax.numpy as jnp
from jax.experimental import pallas as pl


def kernel(z, in_w, in_b, out_w, out_b, codebooks):
    raise NotImplementedError("write your pallas kernel here")


# speedup vs baseline: 1.0402x; 1.0402x over previous
import functools

import jax
import jax.numpy as jnp
from jax.experimental import pallas as pl
from jax.experimental.pallas import tpu as pltpu

N_STAGES = 9
CB_K = 1024
CB_D = 256
D_IN = 512
TB = 512


def _rvq_kernel(res_in_ref, in_wT_ref, in_b_ref, out_wT_ref, out_b_ref,
                cb_ref, cb_nT_ref, cb_sq_ref,
                z_q_ref, codes_ref, lat_ref, loss_ref):
    blk = pl.program_id(0)

    @pl.when(blk == 0)
    def _init():
        loss_ref[...] = jnp.zeros_like(loss_ref)

    res = res_in_ref[...]
    z_q = jnp.zeros((TB, D_IN), jnp.float32)
    loss = jnp.zeros((1, 1), jnp.float32)
    idx_list = []
    for i in range(N_STAGES):
        z_e = jnp.dot(res, in_wT_ref[i]) + in_b_ref[i:i + 1, :]
        n = jnp.sqrt(jnp.sum(z_e * z_e, axis=-1, keepdims=True))
        enc_n = z_e / jnp.maximum(n, 1e-12)
        enc_sq = jnp.sum(enc_n * enc_n, axis=-1, keepdims=True)
        sim = jnp.dot(enc_n, cb_nT_ref[i])
        dist = (enc_sq - 2.0 * sim) + cb_sq_ref[i:i + 1, :]
        idx = jnp.argmax(-dist, axis=1)
        onehot = (idx[:, None] ==
                  jax.lax.broadcasted_iota(jnp.int32, (TB, CB_K), 1)
                  ).astype(jnp.float32)
        z_q_c = jnp.dot(onehot, cb_ref[i],
                        precision=jax.lax.Precision.HIGHEST)
        diff = z_e - z_q_c
        loss = loss + jnp.sum(diff * diff)
        z_q_i = jnp.dot(z_q_c, out_wT_ref[i]) + out_b_ref[i:i + 1, :]
        z_q = z_q + z_q_i
        res = res - z_q_i
        idx_list.append(idx)
        lat_ref[:, i * CB_D:(i + 1) * CB_D] = z_e

    z_q_ref[...] = z_q
    codes_ref[...] = jnp.stack(idx_list, axis=1)
    loss_ref[...] += loss


def kernel(z, in_w, in_b, out_w, out_b, codebooks):
    Bz, D, Tz = z.shape
    BT = Bz * Tz
    nb = BT // TB

    res0 = jnp.transpose(z, (0, 2, 1)).reshape(BT, D)
    in_wT = jnp.transpose(in_w, (0, 2, 1))
    out_wT = jnp.transpose(out_w, (0, 2, 1))
    cb_norm = jnp.sqrt(jnp.sum(codebooks * codebooks, axis=-1, keepdims=True))
    cb_n = codebooks / jnp.maximum(cb_norm, 1e-12)
    cb_nT = jnp.transpose(cb_n, (0, 2, 1))
    cb_sq = jnp.sum(cb_n * cb_n, axis=-1)

    full = lambda *shape: pl.BlockSpec(shape, lambda b: (0,) * len(shape))
    grid_spec = pl.GridSpec(
        grid=(nb,),
        in_specs=[
            pl.BlockSpec((TB, D_IN), lambda b: (b, 0)),
            full(N_STAGES, D_IN, CB_D),
            full(N_STAGES, CB_D),
            full(N_STAGES, CB_D, D_IN),
            full(N_STAGES, D_IN),
            full(N_STAGES, CB_K, CB_D),
            full(N_STAGES, CB_D, CB_K),
            full(N_STAGES, CB_K),
        ],
        out_specs=[
            pl.BlockSpec((TB, D_IN), lambda b: (b, 0)),
            pl.BlockSpec((TB, N_STAGES), lambda b: (b, 0)),
            pl.BlockSpec((TB, N_STAGES * CB_D), lambda b: (b, 0)),
            pl.BlockSpec((1, 1), lambda b: (0, 0)),
        ],
    )
    z_q_t, codes_t, lat_t, loss_sum = pl.pallas_call(
        _rvq_kernel,
        grid_spec=grid_spec,
        out_shape=[
            jax.ShapeDtypeStruct((BT, D_IN), jnp.float32),
            jax.ShapeDtypeStruct((BT, N_STAGES), jnp.int32),
            jax.ShapeDtypeStruct((BT, N_STAGES * CB_D), jnp.float32),
            jax.ShapeDtypeStruct((1, 1), jnp.float32),
        ],
        compiler_params=pltpu.CompilerParams(
            dimension_semantics=("arbitrary",),
        ),
    )(res0, in_wT, in_b, out_wT, out_b, codebooks, cb_nT, cb_sq)

    z_q = jnp.transpose(z_q_t.reshape(Bz, Tz, D), (0, 2, 1))
    codes = jnp.transpose(codes_t.reshape(Bz, Tz, N_STAGES), (0, 2, 1))
    lat = jnp.transpose(
        lat_t.reshape(Bz, Tz, N_STAGES, CB_D), (0, 2, 3, 1)
    ).reshape(Bz, N_STAGES * CB_D, Tz)
    loss = (loss_sum[0, 0] / jnp.float32(BT * CB_D)).reshape(())
    return (z_q, codes, lat, loss, loss)
